# Initial kernel scaffold; baseline (speedup 1.0000x reference)
import jax
import jax.numpy as jnp
from jax.experimental import pallas as pl


def kernel(x, edge_index, edge_attr, W1, b1, W2, b2, Wc1, bc1, Wc2, bc2, Wc3, bc3):
    raise NotImplementedError("write your pallas kernel here")


# speedup vs baseline: 5.7472x; 5.7472x over previous
import functools

import jax
import jax.numpy as jnp
from jax import lax
from jax.experimental import pallas as pl
from jax.experimental.pallas import tpu as pltpu
from jax.experimental.pallas import tpu_sc as plsc

CH = 128


def _deg_body(n_pad, chunks, rem_w, nc, ns, dst_hbm, ones_hbm, zeros_hbm,
              out_hbm, didx, ones_v, deg_sp):
    c = lax.axis_index("c")
    s = lax.axis_index("s")
    w = c * ns + s
    rpt = n_pad // ns
    stride = nc * ns * CH

    pltpu.sync_copy(zeros_hbm.at[pl.ds(s * rpt, rpt)],
                    deg_sp.at[pl.ds(s * rpt, rpt)])
    pltpu.sync_copy(ones_hbm, ones_v)
    plsc.subcore_barrier()

    def do_chunk(base):
        pltpu.sync_copy(dst_hbm.at[pl.ds(base, CH)], didx)
        pltpu.sync_copy(ones_v, deg_sp.at[didx], add=True)

    def chunk_body(j, _):
        do_chunk(j * stride + w * CH)
        return 0
    lax.fori_loop(0, chunks, chunk_body, 0)

    @pl.when(w < rem_w)
    def _():
        do_chunk(chunks * stride + w * CH)

    plsc.subcore_barrier()
    pltpu.sync_copy(deg_sp.at[pl.ds(s * rpt, rpt)],
                    out_hbm.at[c, pl.ds(s * rpt, rpt)])


def _conv_edge_body(n_pad, chunks, rem_w, ns, hh,
                    src_hbm, dst_hbm, lo_hbm, hi_hbm, zeros_hbm, out_hbm,
                    sidx, didx, rows, acc_sp, sem):
    c = lax.axis_index("c")
    s = lax.axis_index("s")
    rpt = n_pad // ns
    stride = ns * CH

    pltpu.sync_copy(zeros_hbm.at[pl.ds(s * rpt, rpt)],
                    acc_sp.at[pl.ds(s * rpt, rpt)])
    plsc.subcore_barrier()

    def edge_loop(tbl):
        def do_chunk(base):
            pltpu.sync_copy(src_hbm.at[pl.ds(base, CH)], sidx)
            pltpu.sync_copy(dst_hbm.at[pl.ds(base, CH)], didx)
            pltpu.async_copy(tbl.at[sidx], rows, sem).wait()
            pltpu.sync_copy(rows, acc_sp.at[didx], add=True)

        def chunk_body(j, _):
            do_chunk(j * stride + s * CH)
            return 0
        lax.fori_loop(0, chunks, chunk_body, 0)

        @pl.when(s < rem_w)
        def _():
            do_chunk(chunks * stride + s * CH)

    @pl.when(c == 0)
    def _():
        edge_loop(lo_hbm)

    @pl.when(c == 1)
    def _():
        edge_loop(hi_hbm)

    plsc.subcore_barrier()
    pltpu.sync_copy(acc_sp.at[pl.ds(s * rpt, rpt)],
                    out_hbm.at[c, pl.ds(s * rpt, rpt)])


def _cls_gather_body(chunks, rem_w, nc, ns,
                     src_hbm, dst_hbm, a_hbm, b_hbm, ga_hbm, gb_hbm,
                     sidx, didx, rows_a, rows_b, sem):
    c = lax.axis_index("c")
    s = lax.axis_index("s")
    w = c * ns + s
    stride = nc * ns * CH

    def do_chunk(base):
        pltpu.sync_copy(src_hbm.at[pl.ds(base, CH)], sidx)
        pltpu.sync_copy(dst_hbm.at[pl.ds(base, CH)], didx)
        pltpu.async_copy(a_hbm.at[sidx], rows_a, sem).wait()
        pltpu.sync_copy(rows_a, ga_hbm.at[pl.ds(base, CH)])
        pltpu.async_copy(b_hbm.at[didx], rows_b, sem).wait()
        pltpu.sync_copy(rows_b, gb_hbm.at[pl.ds(base, CH)])

    def chunk_body(j, _):
        do_chunk(j * stride + w * CH)
        return 0
    lax.fori_loop(0, chunks, chunk_body, 0)

    @pl.when(w < rem_w)
    def _():
        do_chunk(chunks * stride + w * CH)


def _tc_xw_body(dinv_ref, x_ref, w_ref, h_ref, lo_ref, hi_ref):
    hh = lo_ref.shape[1]
    dinv = dinv_ref[...]
    h = jnp.dot(x_ref[...], w_ref[...], preferred_element_type=jnp.float32)
    h_ref[...] = h
    hs = h * dinv
    lo_ref[...] = hs[:, :hh]
    hi_ref[...] = hs[:, hh:]


def _tc_mid_body(dinv_ref, acc_ref, h_ref, b_ref, w_ref,
                 g_ref, lo_ref, hi_ref):
    hh = lo_ref.shape[1]
    dinv = dinv_ref[...]
    acc = jnp.concatenate([acc_ref[0], acc_ref[1]], axis=1)
    out1 = acc * dinv + h_ref[...] * (dinv * dinv) + b_ref[...]
    g = jnp.dot(out1, w_ref[...], preferred_element_type=jnp.float32)
    g_ref[...] = g
    hs = g * dinv
    lo_ref[...] = hs[:, :hh]
    hi_ref[...] = hs[:, hh:]


def _tc_fin_body(dinv_ref, acc_ref, h_ref, b_ref, wa_ref, wb_ref,
                 a_ref, bm_ref):
    dinv = dinv_ref[...]
    acc = jnp.concatenate([acc_ref[0], acc_ref[1]], axis=1)
    h2 = acc * dinv + h_ref[...] * (dinv * dinv) + b_ref[...]
    a_ref[...] = jnp.dot(h2, wa_ref[...], preferred_element_type=jnp.float32)
    bm_ref[...] = jnp.dot(h2, wb_ref[...], preferred_element_type=jnp.float32)


def _tc_mlp_body(ga_ref, gb_ref, ea_ref, wc_ref, bc1_ref, w2_ref, bc2_ref,
                 w3_ref, bc3_ref, out_ref):
    g = (ga_ref[...] + gb_ref[...]
         + jnp.dot(ea_ref[...], wc_ref[...], preferred_element_type=jnp.float32)
         + bc1_ref[...])
    z1 = jnp.tanh(g)
    z2 = jnp.tanh(jnp.dot(z1, w2_ref[...], preferred_element_type=jnp.float32)
                  + bc2_ref[...])
    out_ref[...] = (jnp.dot(z2, w3_ref[...], preferred_element_type=jnp.float32)
                    + bc3_ref[...])


def kernel(x, edge_index, edge_attr, W1, b1, W2, b2,
           Wc1, bc1, Wc2, bc2, Wc3, bc3):
    f32 = jnp.float32
    N, D = x.shape
    E = edge_index.shape[1]
    H = W1.shape[1]
    HH = H // 2
    DE = edge_attr.shape[1]
    HC2 = Wc2.shape[1]

    info = plsc.get_sparse_core_info()
    NC, NS = info.num_cores, info.num_subcores

    NP = -(-N // (NS * 8)) * (NS * 8)

    src = edge_index[0]
    dst = edge_index[1]
    zeros_nh = jnp.zeros((NP, HH), f32)
    b1r = b1.reshape(1, H)
    b2r = b2.reshape(1, H)
    bc1r = bc1.reshape(1, H)
    bc2r = bc2.reshape(1, HC2)
    bc3r = bc3.reshape(1, 1)
    Wc1a = Wc1[:H]
    Wc1b = Wc1[H:2 * H]
    Wc1c = Wc1[2 * H:]

    mesh = plsc.VectorSubcoreMesh(core_axis_name="c", subcore_axis_name="s")

    stride_all = NC * NS * CH
    full2 = E // stride_all
    rem2 = (E - full2 * stride_all) // CH

    DW = HH
    ones_rows = jnp.zeros((CH, DW), f32).at[:, 0].set(1.0)
    deg_parts = pl.kernel(
        functools.partial(_deg_body, NP, full2, rem2, NC, NS),
        mesh=mesh,
        out_type=jax.ShapeDtypeStruct((NC, NP, DW), f32),
        scratch_types=[
            pltpu.VMEM((CH,), jnp.int32),
            pltpu.VMEM((CH, DW), f32),
            pltpu.VMEM_SHARED((NP, DW), f32),
        ],
    )(dst, ones_rows, zeros_nh)

    deg = jnp.sum(deg_parts[:, :N, 0], axis=0) + 1.0
    dinv = lax.rsqrt(deg).reshape(N, 1)

    R = 1000
    grid = (N // R,)
    dinv_spec = pl.BlockSpec((R, 1), lambda i: (i, 0))
    row_spec = pl.BlockSpec((R, H), lambda i: (i, 0))
    half_spec = pl.BlockSpec((R, HH), lambda i: (i, 0))
    full_w = pl.BlockSpec((D, H), lambda i: (0, 0))
    bias_spec = pl.BlockSpec((1, H), lambda i: (0, 0))
    acc_spec = pl.BlockSpec((2, R, HH), lambda i: (0, i, 0))

    h1, hs1_lo, hs1_hi = pl.pallas_call(
        _tc_xw_body,
        grid=grid,
        in_specs=[dinv_spec, pl.BlockSpec((R, D), lambda i: (i, 0)), full_w],
        out_specs=[row_spec, half_spec, half_spec],
        out_shape=[jax.ShapeDtypeStruct((N, H), f32),
                   jax.ShapeDtypeStruct((N, HH), f32),
                   jax.ShapeDtypeStruct((N, HH), f32)],
    )(dinv, x, W1)

    stride_sc = NS * CH
    full1 = E // stride_sc
    rem1 = (E - full1 * stride_sc) // CH

    conv_edge = pl.kernel(
        functools.partial(_conv_edge_body, NP, full1, rem1, NS, HH),
        mesh=mesh,
        out_type=jax.ShapeDtypeStruct((2, NP, HH), f32),
        scratch_types=[
            pltpu.VMEM((CH,), jnp.int32),
            pltpu.VMEM((CH,), jnp.int32),
            pltpu.VMEM((CH, HH), f32),
            pltpu.VMEM_SHARED((NP, HH), f32),
            pltpu.SemaphoreType.DMA,
        ],
    )
    acc1 = conv_edge(src, dst, hs1_lo, hs1_hi, zeros_nh)

    g2, hs2_lo, hs2_hi = pl.pallas_call(
        _tc_mid_body,
        grid=grid,
        in_specs=[dinv_spec, acc_spec, row_spec, bias_spec, full_w],
        out_specs=[row_spec, half_spec, half_spec],
        out_shape=[jax.ShapeDtypeStruct((N, H), f32),
                   jax.ShapeDtypeStruct((N, HH), f32),
                   jax.ShapeDtypeStruct((N, HH), f32)],
    )(dinv, acc1, h1, b1r, W2)

    acc2 = conv_edge(src, dst, hs2_lo, hs2_hi, zeros_nh)

    A, Bm = pl.pallas_call(
        _tc_fin_body,
        grid=grid,
        in_specs=[dinv_spec, acc_spec, row_spec, bias_spec, full_w, full_w],
        out_specs=[row_spec, row_spec],
        out_shape=[jax.ShapeDtypeStruct((N, H), f32),
                   jax.ShapeDtypeStruct((N, H), f32)],
    )(dinv, acc2, g2, b2r, Wc1a, Wc1b)

    ga, gb = pl.kernel(
        functools.partial(_cls_gather_body, full2, rem2, NC, NS),
        mesh=mesh,
        out_type=[jax.ShapeDtypeStruct((E, H), f32),
                  jax.ShapeDtypeStruct((E, H), f32)],
        scratch_types=[
            pltpu.VMEM((CH,), jnp.int32),
            pltpu.VMEM((CH,), jnp.int32),
            pltpu.VMEM((CH, H), f32),
            pltpu.VMEM((CH, H), f32),
            pltpu.SemaphoreType.DMA,
        ],
    )(src, dst, A, Bm)

    EB = 2000
    egrid = (E // EB,)
    logits = pl.pallas_call(
        _tc_mlp_body,
        grid=egrid,
        in_specs=[
            pl.BlockSpec((EB, H), lambda i: (i, 0)),
            pl.BlockSpec((EB, H), lambda i: (i, 0)),
            pl.BlockSpec((EB, DE), lambda i: (i, 0)),
            pl.BlockSpec((DE, H), lambda i: (0, 0)),
            pl.BlockSpec((1, H), lambda i: (0, 0)),
            pl.BlockSpec((H, HC2), lambda i: (0, 0)),
            pl.BlockSpec((1, HC2), lambda i: (0, 0)),
            pl.BlockSpec((HC2, 1), lambda i: (0, 0)),
            pl.BlockSpec((1, 1), lambda i: (0, 0)),
        ],
        out_specs=pl.BlockSpec((EB, 1), lambda i: (i, 0)),
        out_shape=jax.ShapeDtypeStruct((E, 1), f32),
    )(ga, gb, edge_attr, Wc1c, bc1r, Wc2, bc2r, Wc3, bc3r)

    return logits
